# Initial kernel scaffold; baseline (speedup 1.0000x reference)
import jax
import jax.numpy as jnp
from jax.experimental import pallas as pl


def kernel(news_ids, category_ids, delta_days, news_table, cat_table):
    raise NotImplementedError("write your pallas kernel here")


# speedup vs baseline: 2.3086x; 2.3086x over previous
import jax
import jax.numpy as jnp
from jax import lax
from jax.experimental import pallas as pl
from jax.experimental.pallas import tpu as pltpu
from jax.experimental.pallas import tpu_sc as plsc

M = 1024
L = 200
NEWS_DIM = 64
CAT_DIM = 16

NC = 2
NS = 16
NW = NC * NS
DAYS_PER_W = M // NW
IDX_PER_W = DAYS_PER_W * L

NBUF = 2
GROUPS = L // 16
REM = L - GROUPS * 16


def _news_body(news_tab, idxn_hbm, out_hbm, idxn_v, rows, out_v, sems, isem):
    wid = lax.axis_index("s") * NC + lax.axis_index("c")
    base = wid * IDX_PER_W
    idxn_v[pl.ds(IDX_PER_W, 16)] = jnp.zeros((16,), jnp.int32)
    pltpu.async_copy(idxn_hbm.at[pl.ds(base, IDX_PER_W)],
                     idxn_v.at[pl.ds(0, IDX_PER_W)], isem).wait()

    def issue_day(ld, buf):
        off = ld * L

        @plsc.parallel_loop(0, GROUPS * 16, step=16)
        def grp(g):
            ids = idxn_v[pl.ds(off + g, 16)]
            for j in range(16):
                rid = ids[j]
                pltpu.async_copy(news_tab.at[pl.ds(rid, 1)],
                                 rows.at[buf, pl.ds(g + j, 1)], sems[buf])

        ids = idxn_v[pl.ds(off + GROUPS * 16, 16)]
        for j in range(REM):
            rid = ids[j]
            pltpu.async_copy(news_tab.at[pl.ds(rid, 1)],
                             rows.at[buf, pl.ds(GROUPS * 16 + j, 1)],
                             sems[buf])

    def wait_day(buf):
        pltpu.make_async_copy(
            news_tab.at[pl.ds(0, L)], rows.at[buf], sems[buf]).wait()

    def accumulate(ld, buf):
        zero = jnp.zeros((16,), jnp.float32)

        @plsc.parallel_loop(0, L, step=1, unroll=4,
                            carry=(zero, zero, zero, zero))
        def racc(r, acc):
            a0, a1, a2, a3 = acc
            a0 = a0 + rows[buf, r, pl.ds(0, 16)]
            a1 = a1 + rows[buf, r, pl.ds(16, 16)]
            a2 = a2 + rows[buf, r, pl.ds(32, 16)]
            a3 = a3 + rows[buf, r, pl.ds(48, 16)]
            return (a0, a1, a2, a3)

        a0, a1, a2, a3 = racc
        s = jnp.float32(1.0 / L)
        out_v[ld, pl.ds(0, 16)] = a0 * s
        out_v[ld, pl.ds(16, 16)] = a1 * s
        out_v[ld, pl.ds(32, 16)] = a2 * s
        out_v[ld, pl.ds(48, 16)] = a3 * s

    issue_day(0, 0)
    issue_day(1, 1)

    def pair_body(i, carry):
        for buf in range(NBUF):
            ld = i * NBUF + buf
            wait_day(buf)
            accumulate(ld, buf)

            @pl.when(ld + NBUF < DAYS_PER_W)
            def _():
                issue_day(ld + NBUF, buf)
        return carry

    lax.fori_loop(0, DAYS_PER_W // NBUF, pair_body, 0)
    pltpu.sync_copy(out_v, out_hbm.at[pl.ds(wid * DAYS_PER_W, DAYS_PER_W)])


def _cat_body(cat_tab, idxc_hbm, out_hbm, idxc_v, rows_c, out_v, sems):
    wid = lax.axis_index("s") * NC + lax.axis_index("c")
    base = wid * IDX_PER_W
    pltpu.sync_copy(idxc_hbm.at[pl.ds(base, IDX_PER_W)], idxc_v)

    CH = ((0, 128), (128, 72))

    def day_copies(ld, buf):
        off = ld * L
        return [pltpu.make_async_copy(
            cat_tab.at[idxc_v.at[pl.ds(off + c0, cw)]],
            rows_c.at[buf, pl.ds(c0, cw)], sems[buf]) for (c0, cw) in CH]

    def start_day(ld, buf):
        for c in day_copies(ld, buf):
            c.start()

    start_day(0, 0)
    start_day(1, 1)

    def accumulate(ld, buf):
        zero = jnp.zeros((16,), jnp.float32)

        @plsc.parallel_loop(0, L, step=1, unroll=4, carry=(zero,))
        def racc(r, acc):
            return (acc[0] + rows_c[buf, r, pl.ds(0, 16)],)

        out_v[ld, pl.ds(0, 16)] = racc[0] * jnp.float32(1.0 / L)

    def pair_body(i, carry):
        for buf in range(NBUF):
            ld = i * NBUF + buf
            for c in day_copies(ld, buf):
                c.wait()
            accumulate(ld, buf)

            @pl.when(ld + NBUF < DAYS_PER_W)
            def _():
                start_day(ld + NBUF, buf)
        return carry

    lax.fori_loop(0, DAYS_PER_W // NBUF, pair_body, 0)
    pltpu.sync_copy(out_v, out_hbm.at[pl.ds(wid * DAYS_PER_W, DAYS_PER_W)])


@jax.jit
def _pooled(news_ids_flat, category_ids_flat, news_table, cat_table):
    mesh = plsc.VectorSubcoreMesh(core_axis_name="c", subcore_axis_name="s")
    zn = pl.kernel(
        _news_body,
        out_type=jax.ShapeDtypeStruct((M, NEWS_DIM), jnp.float32),
        mesh=mesh,
        compiler_params=pltpu.CompilerParams(use_tc_tiling_on_sc=True),
        scratch_types=[
            pltpu.VMEM((IDX_PER_W + 16,), jnp.int32),
            pltpu.VMEM((NBUF, L, NEWS_DIM), jnp.float32),
            pltpu.VMEM((DAYS_PER_W, NEWS_DIM), jnp.float32),
            [pltpu.SemaphoreType.DMA] * NBUF,
            pltpu.SemaphoreType.DMA,
        ],
    )(news_table, news_ids_flat)
    zc = pl.kernel(
        _cat_body,
        out_type=jax.ShapeDtypeStruct((M, CAT_DIM), jnp.float32),
        mesh=mesh,
        compiler_params=pltpu.CompilerParams(use_tc_tiling_on_sc=False),
        scratch_types=[
            pltpu.VMEM((IDX_PER_W,), jnp.int32),
            pltpu.VMEM((NBUF, L, CAT_DIM), jnp.float32),
            pltpu.VMEM((DAYS_PER_W, CAT_DIM), jnp.float32),
            [pltpu.SemaphoreType.DMA] * NBUF,
        ],
    )(cat_table, category_ids_flat)
    return jnp.concatenate([zn, zc], axis=1)


def kernel(news_ids, category_ids, delta_days, news_table, cat_table):
    idxn = news_ids.reshape(-1).astype(jnp.int32)
    idxc = category_ids.reshape(-1).astype(jnp.int32)
    Z = _pooled(idxn, idxc, news_table, cat_table)
    return (Z, delta_days.astype(jnp.float32))


# speedup vs baseline: 2.3244x; 1.0069x over previous
import jax
import jax.numpy as jnp
from jax import lax
from jax.experimental import pallas as pl
from jax.experimental.pallas import tpu as pltpu
from jax.experimental.pallas import tpu_sc as plsc

M = 1024
L = 200
NEWS_DIM = 64
CAT_DIM = 16
D = NEWS_DIM + CAT_DIM
CATP = 1024

NC = 2
NS = 16
NW = NC * NS
DAYS_PER_W = M // NW
IDX_PER_W = DAYS_PER_W * L

NBUF = 2
GROUPS = L // 16
REM = L - GROUPS * 16


def _body(news_tab, catT, idxn_hbm, idxc_hbm, out_hbm,
          idxn_v, idxc_v, catT_v, rows, tbuf, out_v, sems, isem):
    wid = lax.axis_index("s") * NC + lax.axis_index("c")
    base = wid * IDX_PER_W
    idxn_v[pl.ds(IDX_PER_W, 16)] = jnp.zeros((16,), jnp.int32)
    idxc_v[pl.ds(IDX_PER_W, 16)] = jnp.zeros((16,), jnp.int32)
    c1 = pltpu.async_copy(idxn_hbm.at[pl.ds(base, IDX_PER_W)],
                          idxn_v.at[pl.ds(0, IDX_PER_W)], isem)
    c2 = pltpu.async_copy(idxc_hbm.at[pl.ds(base, IDX_PER_W)],
                          idxc_v.at[pl.ds(0, IDX_PER_W)], isem)
    c3 = pltpu.async_copy(catT, catT_v, isem)
    c1.wait()
    c2.wait()
    c3.wait()

    iota = lax.iota(jnp.int32, 16)
    lane_lt8 = iota < 8

    def issue_day(ld, buf):
        off = ld * L

        @plsc.parallel_loop(0, GROUPS * 16, step=16)
        def grp(g):
            ids = idxn_v[pl.ds(off + g, 16)]
            for j in range(16):
                rid = ids[j]
                pltpu.async_copy(news_tab.at[pl.ds(rid, 1)],
                                 rows.at[buf, pl.ds(g + j, 1)], sems[buf])

        ids = idxn_v[pl.ds(off + GROUPS * 16, 16)]
        for j in range(REM):
            rid = ids[j]
            pltpu.async_copy(news_tab.at[pl.ds(rid, 1)],
                             rows.at[buf, pl.ds(GROUPS * 16 + j, 1)],
                             sems[buf])

    def wait_day(buf):
        pltpu.make_async_copy(
            news_tab.at[pl.ds(0, L)], rows.at[buf], sems[buf]).wait()

    def cat_accumulate(ld):
        off = ld * L
        zero = jnp.zeros((16,), jnp.float32)

        @plsc.parallel_loop(0, GROUPS * 16, step=16,
                            carry=tuple([zero] * 16))
        def cacc(g, acc):
            ids = idxc_v[pl.ds(off + g, 16)]
            return tuple(
                acc[c] + plsc.load_gather(catT_v, [jnp.full((16,), c,
                                                            jnp.int32), ids])
                for c in range(16))

        ids = idxc_v[pl.ds(off + GROUPS * 16, 16)]
        ids = jnp.where(lane_lt8, ids, jnp.int32(1000))
        accs = tuple(
            cacc[c] + plsc.load_gather(catT_v, [jnp.full((16,), c, jnp.int32),
                                                ids])
            for c in range(16))
        for c in range(16):
            tbuf[c, pl.ds(0, 16)] = accs[c]
        tot = jnp.zeros((16,), jnp.float32)
        for l in range(16):
            tot = tot + plsc.load_gather(tbuf, [iota,
                                                jnp.full((16,), l, jnp.int32)])
        out_v[ld, pl.ds(64, 16)] = tot * jnp.float32(1.0 / L)

    def news_accumulate(ld, buf):
        zero = jnp.zeros((16,), jnp.float32)

        @plsc.parallel_loop(0, L, step=1, unroll=4,
                            carry=(zero, zero, zero, zero))
        def racc(r, acc):
            a0, a1, a2, a3 = acc
            a0 = a0 + rows[buf, r, pl.ds(0, 16)]
            a1 = a1 + rows[buf, r, pl.ds(16, 16)]
            a2 = a2 + rows[buf, r, pl.ds(32, 16)]
            a3 = a3 + rows[buf, r, pl.ds(48, 16)]
            return (a0, a1, a2, a3)

        a0, a1, a2, a3 = racc
        s = jnp.float32(1.0 / L)
        out_v[ld, pl.ds(0, 16)] = a0 * s
        out_v[ld, pl.ds(16, 16)] = a1 * s
        out_v[ld, pl.ds(32, 16)] = a2 * s
        out_v[ld, pl.ds(48, 16)] = a3 * s

    issue_day(0, 0)
    issue_day(1, 1)

    def pair_body(i, carry):
        for buf in range(NBUF):
            ld = i * NBUF + buf
            cat_accumulate(ld)
            wait_day(buf)
            news_accumulate(ld, buf)

            @pl.when(ld + NBUF < DAYS_PER_W)
            def _():
                issue_day(ld + NBUF, buf)
        return carry

    lax.fori_loop(0, DAYS_PER_W // NBUF, pair_body, 0)
    pltpu.sync_copy(out_v, out_hbm.at[pl.ds(wid * DAYS_PER_W, DAYS_PER_W)])


@jax.jit
def _pooled(news_ids_flat, category_ids_flat, news_table, catT):
    mesh = plsc.VectorSubcoreMesh(core_axis_name="c", subcore_axis_name="s")
    return pl.kernel(
        _body,
        out_type=jax.ShapeDtypeStruct((M, D), jnp.float32),
        mesh=mesh,
        compiler_params=pltpu.CompilerParams(use_tc_tiling_on_sc=True,
                                             needs_layout_passes=False),
        scratch_types=[
            pltpu.VMEM((IDX_PER_W + 16,), jnp.int32),
            pltpu.VMEM((IDX_PER_W + 16,), jnp.int32),
            pltpu.VMEM((CAT_DIM, CATP), jnp.float32),
            pltpu.VMEM((NBUF, L, NEWS_DIM), jnp.float32),
            pltpu.VMEM((16, 16), jnp.float32),
            pltpu.VMEM((DAYS_PER_W, D), jnp.float32),
            [pltpu.SemaphoreType.DMA] * NBUF,
            pltpu.SemaphoreType.DMA,
        ],
    )(news_table, catT, news_ids_flat, category_ids_flat)


def kernel(news_ids, category_ids, delta_days, news_table, cat_table):
    idxn = news_ids.reshape(-1).astype(jnp.int32)
    idxc = category_ids.reshape(-1).astype(jnp.int32)
    catT = jnp.pad(cat_table.T.astype(jnp.float32),
                   ((0, 0), (0, CATP - cat_table.shape[0])))
    Z = _pooled(idxn, idxc, news_table, catT)
    return (Z, delta_days.astype(jnp.float32))
